# baseline (speedup 1.0000x reference)
import jax
import jax.numpy as jnp
from jax import lax
from jax.experimental import pallas as pl
from jax.experimental.pallas import tpu as pltpu
from jax.experimental.pallas import tpu_sc as plsc

B = 16384
NF = 26
V = 100000
D = 32
C = 16

NC = 2
NS = 16
L = 16
NW = NC * NS
RPW = (NF * D) // NW
NB = B // NW
G = 128
NG = B // G


def _tab_kernel(tab_hbm, xT_hbm, xcont_hbm, gamma_hbm, beta_hbm,
                out_hbm, parts_hbm,
                xv, gidx, sidx, gbuf, xc, xcf, pbuf, pacc, gv, bv,
                sem, sem_s):
    c = lax.axis_index("c")
    s = lax.axis_index("s")
    wid = s * NC + c

    pltpu.sync_copy(xcont_hbm.at[pl.ds(s * (NB * NC), NB * NC)], xc)

    def acc_body(i, carry):
        acc, acc2 = carry
        v = xc[i, :]
        return acc + v, acc2 + v * v

    zero = jnp.zeros((L,), jnp.float32)
    acc, acc2 = lax.fori_loop(0, NB * NC, acc_body, (zero, zero))
    pbuf[0, :] = acc
    pbuf[1, :] = acc2
    pltpu.sync_copy(pbuf, parts_hbm.at[c, s])
    plsc.subcore_barrier()

    pltpu.sync_copy(parts_hbm.at[c], pacc)
    tot = jnp.zeros((L,), jnp.float32)
    tot2 = jnp.zeros((L,), jnp.float32)
    for k in range(NS):
        tot = tot + pacc[k, 0, :]
        tot2 = tot2 + pacc[k, 1, :]
    inv_b = jnp.float32(1.0 / B)
    mean = tot * inv_b
    var = tot2 * inv_b - mean * mean
    x = var + jnp.float32(1e-5)
    i32 = lax.bitcast_convert_type(x, jnp.int32)
    i32 = jnp.int32(0x5F3759DF) - lax.shift_right_logical(i32, 1)
    y = lax.bitcast_convert_type(i32, jnp.float32)
    for _ in range(4):
        y = y * (jnp.float32(1.5) - jnp.float32(0.5) * x * y * y)
    pltpu.sync_copy(gamma_hbm, gv)
    pltpu.sync_copy(beta_hbm, bv)
    scale = gv[...] * y
    shift = bv[...] - mean * scale

    iota = lax.iota(jnp.int32, L)
    bn_base = wid * NB * (NF * D + C) + NF * D

    def bn_body(i, _):
        xcf[pl.ds(i * L, L)] = xc[c * NB + i, :] * scale + shift
        sidx[pl.ds(i * L, L)] = iota + (bn_base + i * (NF * D + C))
        return 0

    lax.fori_loop(0, NB, bn_body, 0)

    def bn_fire(g, _):
        pltpu.async_copy(xcf.at[pl.ds(g * G, G)],
                         out_hbm.at[sidx.at[pl.ds(g * G, G)]], sem_s)
        return 0

    lax.fori_loop(0, (NB * C) // G, bn_fire, 0)
    pltpu.make_async_copy(tab_hbm.at[pl.ds(0, NB * C)], xcf, sem_s).wait()

    r0 = wid * RPW
    f1 = r0 // D
    n1 = jnp.minimum(RPW, (f1 + 1) * D - r0)

    iota848 = iota * (NF * D + C)

    def row_work(t, r):
        slot = lax.rem(t, 2)

        so = slot * B

        @pl.when(t >= 2)
        def _drain():
            pltpu.make_async_copy(tab_hbm.at[pl.ds(0, B)],
                                  gbuf.at[pl.ds(so, B)], sem_s).wait()

        rv = r * V

        def idx_body(i, _):
            gidx[pl.ds(i * L, L)] = xv[pl.ds(i * L, L)] + rv
            sidx[pl.ds(so + i * L, L)] = \
                iota848 + (i * (L * (NF * D + C)) + r)
            return 0

        lax.fori_loop(0, B // L, idx_body, 0)

        def fire_body(g, _):
            pltpu.async_copy(tab_hbm.at[gidx.at[pl.ds(g * G, G)]],
                             gbuf.at[pl.ds(so + g * G, G)], sem)
            return 0

        lax.fori_loop(0, NG, fire_body, 0)
        pltpu.make_async_copy(tab_hbm.at[pl.ds(0, B)],
                              gbuf.at[pl.ds(so, B)], sem).wait()

        def scat_body(g, _):
            pltpu.async_copy(gbuf.at[pl.ds(so + g * G, G)],
                             out_hbm.at[sidx.at[pl.ds(so + g * G, G)]],
                             sem_s)
            return 0

        lax.fori_loop(0, NG, scat_body, 0)
        return 0

    def seg(f, lo, toff, n):
        @pl.when(n > 0)
        def _():
            pltpu.sync_copy(xT_hbm.at[f], xv)

            def body(k, _):
                return row_work(toff + k, lo + k)

            lax.fori_loop(0, n, body, 0)

    seg(f1, r0, 0, n1)
    seg(f1 + 1, r0 + n1, n1, RPW - n1)

    pltpu.make_async_copy(tab_hbm.at[pl.ds(0, B)],
                          gbuf.at[pl.ds(0, B)], sem_s).wait()
    pltpu.make_async_copy(tab_hbm.at[pl.ds(0, B)],
                          gbuf.at[pl.ds(B, B)], sem_s).wait()


@jax.jit
def _run(tabflat, xT, x_cont, gamma, beta):
    mesh = plsc.VectorSubcoreMesh(core_axis_name="c", subcore_axis_name="s")
    f = pl.kernel(
        _tab_kernel,
        mesh=mesh,
        compiler_params=pltpu.CompilerParams(use_tc_tiling_on_sc=False),
        out_type=[
            jax.ShapeDtypeStruct((B * (NF * D + C),), jnp.float32),
            jax.ShapeDtypeStruct((NC, NS, 2, L), jnp.float32),
        ],
        scratch_types=[
            pltpu.VMEM((B,), jnp.int32),
            pltpu.VMEM((B,), jnp.int32),
            pltpu.VMEM((2 * B,), jnp.int32),
            pltpu.VMEM((2 * B,), jnp.float32),
            pltpu.VMEM((NB * NC, C), jnp.float32),
            pltpu.VMEM((NB * C,), jnp.float32),
            pltpu.VMEM((2, L), jnp.float32),
            pltpu.VMEM((NS, 2, L), jnp.float32),
            pltpu.VMEM((L,), jnp.float32),
            pltpu.VMEM((L,), jnp.float32),
            pltpu.SemaphoreType.DMA,
            pltpu.SemaphoreType.DMA,
        ],
    )
    out, _ = f(tabflat, xT, x_cont, gamma, beta)
    return out.reshape(B, NF * D + C)


def kernel(x_cat, x_cont, emb_tables, bn_gamma, bn_beta):
    tabflat = emb_tables.transpose(0, 2, 1).reshape(NF * D * V)
    xT = x_cat.astype(jnp.int32).T
    return _run(tabflat, xT, x_cont, bn_gamma, bn_beta)
